# Initial kernel scaffold; baseline (speedup 1.0000x reference)
import jax
import jax.numpy as jnp
from jax.experimental import pallas as pl


def kernel(input_ids, weight_quantized, scale, zero_point):
    raise NotImplementedError("write your pallas kernel here")


# speedup vs baseline: 1.1112x; 1.1112x over previous
import jax
import jax.numpy as jnp
from jax import lax
from jax.experimental import pallas as pl
from jax.experimental.pallas import tpu as pltpu
from jax.experimental.pallas import tpu_sc as plsc

NUM_EMBEDDINGS = 100000
EMBEDDING_DIM = 128
WORDS = EMBEDDING_DIM // 4
NW = 32
B_TOTAL = 4096 * 50
PER_W = B_TOTAL // NW
GROUP = 256
N_GROUPS = PER_W // GROUP
IDX_ROWS_PER_W = PER_W // 128


def _body(ids_hbm, table_hbm, params_hbm, out_hbm,
          idx_v, rows_v, out_v, lut_v, par_v, sem):
    c = lax.axis_index("c")
    s = lax.axis_index("s")
    wid = s * 2 + c

    pltpu.sync_copy(params_hbm, par_v)
    pltpu.sync_copy(ids_hbm.at[pl.ds(pl.multiple_of(wid * PER_W, 8), PER_W)],
                    idx_v)

    scv = par_v[pl.ds(0, 16)]
    zpv = par_v[pl.ds(16, 16)]
    ii = lax.iota(jnp.int32, 16)

    for k in range(16):
        q = (ii + 16 * k).astype(jnp.float32)
        lut_v[pl.ds(16 * k, 16)] = (q - zpv) * scv

    out_base = pl.multiple_of(wid * (PER_W * EMBEDDING_DIM), 8)

    for g in range(N_GROUPS):
        cps = [
            pltpu.async_copy(table_hbm.at[idx_v.at[pl.ds(g * GROUP + 128 * h, 128)]],
                             rows_v.at[pl.ds(128 * h, 128)], sem)
            for h in range(2)
        ]
        for cp in cps:
            cp.wait()

        @pl.loop(0, GROUP)
        def _(r):
            for gg in range(2):
                w = rows_v[r, pl.ds(16 * gg, 16)]
                base = r * EMBEDDING_DIM + 64 * gg + 4 * ii
                for j in range(4):
                    b = lax.shift_right_logical(w, j * 8) & 0xFF if j else w & 0xFF
                    val = plsc.load_gather(lut_v, [b])
                    plsc.store_scatter(out_v, [base + j], val)

        pltpu.sync_copy(
            out_v,
            out_hbm.at[pl.ds(out_base + g * (GROUP * EMBEDDING_DIM),
                             GROUP * EMBEDDING_DIM)])


_gather_dequant = pl.kernel(
    _body,
    out_type=jax.ShapeDtypeStruct((B_TOTAL * EMBEDDING_DIM,), jnp.float32),
    mesh=plsc.VectorSubcoreMesh(core_axis_name="c", subcore_axis_name="s"),
    compiler_params=pltpu.CompilerParams(needs_layout_passes=False,
                                         use_tc_tiling_on_sc=False),
    scratch_types=[
        pltpu.VMEM((PER_W,), jnp.int32),
        pltpu.VMEM((GROUP, WORDS), jnp.int32),
        pltpu.VMEM((GROUP * EMBEDDING_DIM,), jnp.float32),
        pltpu.VMEM((256,), jnp.float32),
        pltpu.VMEM((32,), jnp.float32),
        pltpu.SemaphoreType.DMA,
    ],
)


def kernel(input_ids, weight_quantized, scale, zero_point):
    ids = input_ids.reshape(B_TOTAL)
    table = lax.bitcast_convert_type(
        weight_quantized.reshape(NUM_EMBEDDINGS, WORDS, 4), jnp.int32)
    params = jnp.concatenate([
        jnp.broadcast_to(scale.astype(jnp.float32), (16,)),
        jnp.broadcast_to(zero_point.astype(jnp.float32), (16,)),
    ])
    out = _gather_dequant(ids, table, params)
    return out.reshape(4096, 50, EMBEDDING_DIM)


# speedup vs baseline: 1.1861x; 1.0675x over previous
import jax
import jax.numpy as jnp
from jax import lax
from jax.experimental import pallas as pl
from jax.experimental.pallas import tpu as pltpu
from jax.experimental.pallas import tpu_sc as plsc

NUM_EMBEDDINGS = 100000
EMBEDDING_DIM = 128
WORDS = EMBEDDING_DIM // 4
NW = 32
B_TOTAL = 4096 * 50
PER_W = B_TOTAL // NW
GROUP = 256
N_GROUPS = PER_W // GROUP
IDX_ROWS_PER_W = PER_W // 128


def _body(ids_hbm, table_hbm, params_hbm, out_hbm,
          idx_v, rows_v, out_v, lut_v, par_v,
          sem_g0, sem_g1, sem_o0, sem_o1):
    c = lax.axis_index("c")
    s = lax.axis_index("s")
    wid = s * 2 + c
    sem_g = (sem_g0, sem_g1)
    sem_o = (sem_o0, sem_o1)

    pltpu.sync_copy(params_hbm, par_v)
    pltpu.sync_copy(ids_hbm.at[pl.ds(pl.multiple_of(wid * PER_W, 8), PER_W)],
                    idx_v)

    scv = par_v[pl.ds(0, 16)]
    zpv = par_v[pl.ds(16, 16)]
    ii = lax.iota(jnp.int32, 16)

    for k in range(16):
        q = (ii + 16 * k).astype(jnp.float32)
        lut_v[pl.ds(16 * k, 16)] = (q - zpv) * scv

    out_base = pl.multiple_of(wid * (PER_W * EMBEDDING_DIM), 8)

    def fire_gathers(g, b):
        return [
            pltpu.async_copy(
                table_hbm.at[idx_v.at[pl.ds(g * GROUP + 128 * h, 128)]],
                rows_v.at[b, pl.ds(128 * h, 128)], sem_g[b])
            for h in range(2)
        ]

    cps = fire_gathers(0, 0)
    ocp = [None, None]
    for g in range(N_GROUPS):
        b = g % 2
        ncps = fire_gathers(g + 1, (g + 1) % 2) if g + 1 < N_GROUPS else []
        for cp in cps:
            cp.wait()
        if ocp[b] is not None:
            ocp[b].wait()

        @pl.loop(0, GROUP)
        def _(r):
            for gg in range(2):
                w = rows_v[b, r, pl.ds(16 * gg, 16)]
                base = (b * GROUP + r) * EMBEDDING_DIM + 64 * gg + 4 * ii
                for j in range(4):
                    q = lax.shift_right_logical(w, j * 8) & 0xFF if j else w & 0xFF
                    val = plsc.load_gather(lut_v, [q])
                    plsc.store_scatter(out_v, [base + j], val)

        ocp[b] = pltpu.async_copy(
            out_v.at[pl.ds(b * (GROUP * EMBEDDING_DIM), GROUP * EMBEDDING_DIM)],
            out_hbm.at[pl.ds(out_base + g * (GROUP * EMBEDDING_DIM),
                             GROUP * EMBEDDING_DIM)],
            sem_o[b])
        cps = ncps
    for cp in ocp:
        if cp is not None:
            cp.wait()


_gather_dequant = pl.kernel(
    _body,
    out_type=jax.ShapeDtypeStruct((B_TOTAL * EMBEDDING_DIM,), jnp.float32),
    mesh=plsc.VectorSubcoreMesh(core_axis_name="c", subcore_axis_name="s"),
    compiler_params=pltpu.CompilerParams(needs_layout_passes=False,
                                         use_tc_tiling_on_sc=False),
    scratch_types=[
        pltpu.VMEM((PER_W,), jnp.int32),
        pltpu.VMEM((2, GROUP, WORDS), jnp.int32),
        pltpu.VMEM((2 * GROUP * EMBEDDING_DIM,), jnp.float32),
        pltpu.VMEM((256,), jnp.float32),
        pltpu.VMEM((32,), jnp.float32),
        pltpu.SemaphoreType.DMA,
        pltpu.SemaphoreType.DMA,
        pltpu.SemaphoreType.DMA,
        pltpu.SemaphoreType.DMA,
    ],
)


def kernel(input_ids, weight_quantized, scale, zero_point):
    ids = input_ids.reshape(B_TOTAL)
    table = lax.bitcast_convert_type(
        weight_quantized.reshape(NUM_EMBEDDINGS, WORDS, 4), jnp.int32)
    params = jnp.concatenate([
        jnp.broadcast_to(scale.astype(jnp.float32), (16,)),
        jnp.broadcast_to(zero_point.astype(jnp.float32), (16,)),
    ])
    out = _gather_dequant(ids, table, params)
    return out.reshape(4096, 50, EMBEDDING_DIM)


# speedup vs baseline: 2.5131x; 2.1188x over previous
import jax
import jax.numpy as jnp
from jax import lax
from jax.experimental import pallas as pl
from jax.experimental.pallas import tpu as pltpu
from jax.experimental.pallas import tpu_sc as plsc

NUM_EMBEDDINGS = 100000
EMBEDDING_DIM = 128
WORDS = EMBEDDING_DIM // 4
NW = 32
BATCH = 4096
SEQ = 50
B_TOTAL = BATCH * SEQ
BPW = BATCH // NW
PER_W = BPW * SEQ
OUT_TILE = BPW * EMBEDDING_DIM

RP_ROWS = 800
RP_BLOCKS = NUM_EMBEDDINGS // RP_ROWS


def _repack_body(in_ref, out_ref):
    q = pltpu.bitcast(in_ref[...], jnp.int32)
    m = lax.broadcasted_iota(jnp.int32, q.shape, 1) // WORDS
    sh = m * 8
    acc = None
    for k in range(4):
        s_k = jnp.concatenate([q[:, WORDS * k:WORDS * (k + 1)]] * 4, axis=1)
        p_k = lax.shift_right_logical(s_k, sh) & 0xFF
        acc = p_k if k == 0 else acc | (p_k << (8 * k))
    out_ref[...] = acc


_repack = pl.pallas_call(
    _repack_body,
    grid=(RP_BLOCKS,),
    in_specs=[pl.BlockSpec((RP_ROWS, EMBEDDING_DIM), lambda i: (i, 0))],
    out_specs=pl.BlockSpec((RP_ROWS // 4, EMBEDDING_DIM), lambda i: (i, 0)),
    out_shape=jax.ShapeDtypeStruct((NUM_EMBEDDINGS // 4, EMBEDDING_DIM),
                                   jnp.int32),
)


def _body(ids_hbm, table_hbm, params_hbm, out_hbm,
          idx_v, idx_t, rows_v, out_v, lut_v, par_v,
          sem_g0, sem_g1, sem_o0, sem_o1):
    c = lax.axis_index("c")
    s_ax = lax.axis_index("s")
    wid = s_ax * 2 + c
    sem_g = (sem_g0, sem_g1)
    sem_o = (sem_o0, sem_o1)

    pltpu.sync_copy(params_hbm, par_v)
    pltpu.sync_copy(ids_hbm.at[pl.ds(pl.multiple_of(wid * PER_W, 8), PER_W)],
                    idx_v)

    scv = par_v[pl.ds(0, 16)]
    zpv = par_v[pl.ds(16, 16)]
    ii = lax.iota(jnp.int32, 16)

    for k in range(16):
        qv = (ii + 16 * k).astype(jnp.float32)
        lut_v[pl.ds(16 * k, 16)] = (qv - zpv) * scv

    ii50 = ii * SEQ

    @pl.loop(0, SEQ)
    def _(s):
        for cc in range(BPW // 16):
            v = plsc.load_gather(idx_v, [ii50 + (16 * cc * SEQ + s)])
            idx_t[pl.ds(s * BPW + 16 * cc, 16)] = v

    def fire_gather(s, b):
        return pltpu.async_copy(
            table_hbm.at[idx_t.at[pl.ds(s * BPW, BPW)]],
            rows_v.at[b], sem_g[b])

    def dequant_tile(b):
        @pl.loop(0, BPW, unroll=2)
        def _(r):
            base = (b * BPW + r) * EMBEDDING_DIM
            for h in range(2):
                w = rows_v[b, r, pl.ds(16 * h, 16)]
                for k in range(4):
                    qb = lax.shift_right_logical(w, k * 8) & 0xFF if k else w & 0xFF
                    val = plsc.load_gather(lut_v, [qb])
                    out_v[pl.ds(base + WORDS * k + 16 * h, 16)] = val

    def write_out(s, b):
        return pltpu.async_copy(
            out_v.at[pl.ds(b * OUT_TILE, OUT_TILE)],
            out_hbm.at[pl.ds(
                pl.multiple_of(wid * OUT_TILE, 8) + s * (BATCH * EMBEDDING_DIM),
                OUT_TILE)],
            sem_o[b])

    cps = [fire_gather(0, 0), fire_gather(1, 1)]
    ocp = [None, None]
    for s in range(SEQ):
        b = s % 2
        cps[b].wait()
        if ocp[b] is not None:
            ocp[b].wait()
        dequant_tile(b)
        if s + 2 < SEQ:
            cps[b] = fire_gather(s + 2, b)
        ocp[b] = write_out(s, b)
    for cpo in ocp:
        if cpo is not None:
            cpo.wait()


_gather_dequant = pl.kernel(
    _body,
    out_type=jax.ShapeDtypeStruct((B_TOTAL * EMBEDDING_DIM,), jnp.float32),
    mesh=plsc.VectorSubcoreMesh(core_axis_name="c", subcore_axis_name="s"),
    compiler_params=pltpu.CompilerParams(needs_layout_passes=False,
                                         use_tc_tiling_on_sc=False),
    scratch_types=[
        pltpu.VMEM((PER_W,), jnp.int32),
        pltpu.VMEM((PER_W,), jnp.int32),
        pltpu.VMEM((2, BPW, WORDS), jnp.int32),
        pltpu.VMEM((2 * OUT_TILE,), jnp.float32),
        pltpu.VMEM((256,), jnp.float32),
        pltpu.VMEM((32,), jnp.float32),
        pltpu.SemaphoreType.DMA,
        pltpu.SemaphoreType.DMA,
        pltpu.SemaphoreType.DMA,
        pltpu.SemaphoreType.DMA,
    ],
)


def kernel(input_ids, weight_quantized, scale, zero_point):
    ids = input_ids.reshape(B_TOTAL)
    table = _repack(weight_quantized).reshape(NUM_EMBEDDINGS, WORDS)
    params = jnp.concatenate([
        jnp.broadcast_to(scale.astype(jnp.float32), (16,)),
        jnp.broadcast_to(zero_point.astype(jnp.float32), (16,)),
    ])
    out = _gather_dequant(ids, table, params)
    return out.reshape(SEQ, BATCH, EMBEDDING_DIM).transpose(1, 0, 2)


# speedup vs baseline: 3.2655x; 1.2994x over previous
import jax
import jax.numpy as jnp
from jax import lax
from jax.experimental import pallas as pl
from jax.experimental.pallas import tpu as pltpu
from jax.experimental.pallas import tpu_sc as plsc

NUM_EMBEDDINGS = 100000
EMBEDDING_DIM = 128
WORDS = EMBEDDING_DIM // 4
NW = 32
BATCH = 4096
SEQ = 50
B_TOTAL = BATCH * SEQ
BPW = BATCH // NW
PER_W = BPW * SEQ
OUT_TILE = BPW * EMBEDDING_DIM

RP_ROWS = 800
RP_BLOCKS = NUM_EMBEDDINGS // RP_ROWS


def _repack_body(in_ref, out_ref):
    q = pltpu.bitcast(in_ref[...], jnp.int32)
    m = lax.broadcasted_iota(jnp.int32, q.shape, 1) // WORDS
    sh = m * 8
    acc = None
    for k in range(4):
        s_k = jnp.concatenate([q[:, WORDS * k:WORDS * (k + 1)]] * 4, axis=1)
        p_k = lax.shift_right_logical(s_k, sh) & 0xFF
        acc = p_k if k == 0 else acc | (p_k << (8 * k))
    out_ref[...] = acc


_repack = pl.pallas_call(
    _repack_body,
    grid=(RP_BLOCKS,),
    in_specs=[pl.BlockSpec((RP_ROWS, EMBEDDING_DIM), lambda i: (i, 0))],
    out_specs=pl.BlockSpec((RP_ROWS // 4, EMBEDDING_DIM), lambda i: (i, 0)),
    out_shape=jax.ShapeDtypeStruct((NUM_EMBEDDINGS // 4, EMBEDDING_DIM),
                                   jnp.int32),
)


def _body(ids_hbm, table_hbm, params_hbm, out_hbm,
          idx_v, idx_t, rows_v, out_v, lut_v, par_v,
          sem_g0, sem_g1, sem_o0, sem_o1):
    c = lax.axis_index("c")
    s_ax = lax.axis_index("s")
    wid = s_ax * 2 + c
    sem_g = (sem_g0, sem_g1)
    sem_o = (sem_o0, sem_o1)

    pltpu.sync_copy(params_hbm, par_v)
    pltpu.sync_copy(ids_hbm.at[pl.ds(pl.multiple_of(wid * PER_W, 8), PER_W)],
                    idx_v)

    scv = par_v[pl.ds(0, 16)]
    zpv = par_v[pl.ds(16, 16)]
    ii = lax.iota(jnp.int32, 16)

    for k in range(16):
        qv = (ii + 16 * k).astype(jnp.float32)
        lut_v[pl.ds(16 * k, 16)] = (qv - zpv) * scv

    ii50 = ii * SEQ

    @pl.loop(0, SEQ)
    def _(s):
        for cc in range(BPW // 16):
            v = plsc.load_gather(idx_v, [ii50 + (16 * cc * SEQ + s)])
            idx_t[pl.ds(s * BPW + 16 * cc, 16)] = v

    def gather_copy(s, b):
        return pltpu.make_async_copy(
            table_hbm.at[idx_t.at[pl.ds(s * BPW, BPW)]],
            rows_v.at[b], sem_g[b])

    def dequant_tile(b):
        @pl.loop(0, BPW, unroll=4)
        def _(r):
            base = (b * BPW + r) * EMBEDDING_DIM
            for h in range(2):
                w = rows_v[b, r, pl.ds(16 * h, 16)]
                for k in range(4):
                    qb = lax.shift_right_logical(w, k * 8) if k else w
                    if k in (1, 2):
                        val = plsc.load_gather(lut_v, [qb & 0xFF])
                    else:
                        qf = (qb & 0xFF if k == 0 else qb).astype(jnp.float32)
                        val = (qf - zpv) * scv
                    out_v[pl.ds(base + WORDS * k + 16 * h, 16)] = val

    def out_copy(s, b):
        return pltpu.make_async_copy(
            out_v.at[pl.ds(b * OUT_TILE, OUT_TILE)],
            out_hbm.at[pl.ds(
                pl.multiple_of(wid * OUT_TILE, 8) + s * (BATCH * EMBEDDING_DIM),
                OUT_TILE)],
            sem_o[b])

    gather_copy(0, 0).start()
    gather_copy(1, 1).start()

    @pl.loop(0, SEQ, step=2)
    def _(s):
        for par in range(2):
            sb = s + par
            gather_copy(sb, par).wait()

            @pl.when(sb >= 2)
            def _():
                out_copy(sb - 2, par).wait()

            dequant_tile(par)

            @pl.when(sb + 2 < SEQ)
            def _():
                gather_copy(sb + 2, par).start()

            out_copy(sb, par).start()

    out_copy(SEQ - 2, 0).wait()
    out_copy(SEQ - 1, 1).wait()


_gather_dequant = pl.kernel(
    _body,
    out_type=jax.ShapeDtypeStruct((B_TOTAL * EMBEDDING_DIM,), jnp.float32),
    mesh=plsc.VectorSubcoreMesh(core_axis_name="c", subcore_axis_name="s"),
    compiler_params=pltpu.CompilerParams(needs_layout_passes=False,
                                         use_tc_tiling_on_sc=False),
    scratch_types=[
        pltpu.VMEM((PER_W,), jnp.int32),
        pltpu.VMEM((PER_W,), jnp.int32),
        pltpu.VMEM((2, BPW, WORDS), jnp.int32),
        pltpu.VMEM((2 * OUT_TILE,), jnp.float32),
        pltpu.VMEM((256,), jnp.float32),
        pltpu.VMEM((32,), jnp.float32),
        pltpu.SemaphoreType.DMA,
        pltpu.SemaphoreType.DMA,
        pltpu.SemaphoreType.DMA,
        pltpu.SemaphoreType.DMA,
    ],
)


def kernel(input_ids, weight_quantized, scale, zero_point):
    ids = input_ids.reshape(B_TOTAL)
    table = _repack(weight_quantized).reshape(NUM_EMBEDDINGS, WORDS)
    params = jnp.concatenate([
        jnp.broadcast_to(scale.astype(jnp.float32), (16,)),
        jnp.broadcast_to(zero_point.astype(jnp.float32), (16,)),
    ])
    out = _gather_dequant(ids, table, params)
    return out.reshape(SEQ, BATCH, EMBEDDING_DIM).transpose(1, 0, 2)


# speedup vs baseline: 3.2792x; 1.0042x over previous
import jax
import jax.numpy as jnp
from jax import lax
from jax.experimental import pallas as pl
from jax.experimental.pallas import tpu as pltpu
from jax.experimental.pallas import tpu_sc as plsc

NUM_EMBEDDINGS = 100000
EMBEDDING_DIM = 128
WORDS = EMBEDDING_DIM // 4
NW = 32
BATCH = 4096
SEQ = 50
B_TOTAL = BATCH * SEQ
BPW = BATCH // NW
PER_W = BPW * SEQ
OUT_TILE = BPW * EMBEDDING_DIM

RP_ROWS = 800
RP_BLOCKS = NUM_EMBEDDINGS // RP_ROWS


def _repack_body(in_ref, out_ref):
    q = pltpu.bitcast(in_ref[...], jnp.int32)
    m = lax.broadcasted_iota(jnp.int32, q.shape, 1) // WORDS
    sh = m * 8
    acc = None
    for k in range(4):
        s_k = jnp.concatenate([q[:, WORDS * k:WORDS * (k + 1)]] * 4, axis=1)
        p_k = lax.shift_right_logical(s_k, sh) & 0xFF
        acc = p_k if k == 0 else acc | (p_k << (8 * k))
    out_ref[...] = acc


_repack = pl.pallas_call(
    _repack_body,
    grid=(RP_BLOCKS,),
    in_specs=[pl.BlockSpec((RP_ROWS, EMBEDDING_DIM), lambda i: (i, 0))],
    out_specs=pl.BlockSpec((RP_ROWS // 4, EMBEDDING_DIM), lambda i: (i, 0)),
    out_shape=jax.ShapeDtypeStruct((NUM_EMBEDDINGS // 4, EMBEDDING_DIM),
                                   jnp.int32),
)


def _body(ids_hbm, table_hbm, params_hbm, out_hbm,
          idx_t, rows_v, out_v, lut_v, par_v,
          sem_g0, sem_g1, sem_o0, sem_o1):
    c = lax.axis_index("c")
    s_ax = lax.axis_index("s")
    wid = s_ax * 2 + c
    sem_g = (sem_g0, sem_g1)
    sem_o = (sem_o0, sem_o1)

    pltpu.sync_copy(params_hbm, par_v)
    pltpu.sync_copy(ids_hbm.at[:, pl.ds(pl.multiple_of(wid * BPW, 8), BPW)],
                    idx_t)

    scv = par_v[pl.ds(0, 16)]
    zpv = par_v[pl.ds(16, 16)]
    ii = lax.iota(jnp.int32, 16)

    for k in range(16):
        qv = (ii + 16 * k).astype(jnp.float32)
        lut_v[pl.ds(16 * k, 16)] = (qv - zpv) * scv

    def gather_copy(s, b):
        return pltpu.make_async_copy(
            table_hbm.at[idx_t.at[s]],
            rows_v.at[b], sem_g[b])

    def dequant_tile(b):
        @pl.loop(0, BPW, unroll=8)
        def _(r):
            base = (b * BPW + r) * EMBEDDING_DIM
            for h in range(2):
                w = rows_v[b, r, pl.ds(16 * h, 16)]
                for k in range(4):
                    qb = lax.shift_right_logical(w, k * 8) if k else w
                    if k in (1, 2):
                        val = plsc.load_gather(lut_v, [qb & 0xFF])
                    else:
                        qf = (qb & 0xFF if k == 0 else qb).astype(jnp.float32)
                        val = (qf - zpv) * scv
                    out_v[pl.ds(base + WORDS * k + 16 * h, 16)] = val

    def out_copy(s, b):
        return pltpu.make_async_copy(
            out_v.at[pl.ds(b * OUT_TILE, OUT_TILE)],
            out_hbm.at[pl.ds(
                pl.multiple_of(wid * OUT_TILE, 8) + s * (BATCH * EMBEDDING_DIM),
                OUT_TILE)],
            sem_o[b])

    gather_copy(0, 0).start()
    gather_copy(1, 1).start()

    @pl.loop(0, SEQ, step=2)
    def _(s):
        for par in range(2):
            sb = s + par
            gather_copy(sb, par).wait()

            @pl.when(sb >= 2)
            def _():
                out_copy(sb - 2, par).wait()

            dequant_tile(par)

            @pl.when(sb + 2 < SEQ)
            def _():
                gather_copy(sb + 2, par).start()

            out_copy(sb, par).start()

    out_copy(SEQ - 2, 0).wait()
    out_copy(SEQ - 1, 1).wait()


_gather_dequant = pl.kernel(
    _body,
    out_type=jax.ShapeDtypeStruct((B_TOTAL * EMBEDDING_DIM,), jnp.float32),
    mesh=plsc.VectorSubcoreMesh(core_axis_name="c", subcore_axis_name="s"),
    compiler_params=pltpu.CompilerParams(needs_layout_passes=False,
                                         use_tc_tiling_on_sc=False),
    scratch_types=[
        pltpu.VMEM((SEQ, BPW), jnp.int32),
        pltpu.VMEM((2, BPW, WORDS), jnp.int32),
        pltpu.VMEM((2 * OUT_TILE,), jnp.float32),
        pltpu.VMEM((256,), jnp.float32),
        pltpu.VMEM((32,), jnp.float32),
        pltpu.SemaphoreType.DMA,
        pltpu.SemaphoreType.DMA,
        pltpu.SemaphoreType.DMA,
        pltpu.SemaphoreType.DMA,
    ],
)


def kernel(input_ids, weight_quantized, scale, zero_point):
    ids = input_ids.T
    table = _repack(weight_quantized).reshape(NUM_EMBEDDINGS, WORDS)
    params = jnp.concatenate([
        jnp.broadcast_to(scale.astype(jnp.float32), (16,)),
        jnp.broadcast_to(zero_point.astype(jnp.float32), (16,)),
    ])
    out = _gather_dequant(ids, table, params)
    return out.reshape(SEQ, BATCH, EMBEDDING_DIM).transpose(1, 0, 2)


# speedup vs baseline: 3.2863x; 1.0022x over previous
import jax
import jax.numpy as jnp
from jax import lax
from jax.experimental import pallas as pl
from jax.experimental.pallas import tpu as pltpu
from jax.experimental.pallas import tpu_sc as plsc

NUM_EMBEDDINGS = 100000
EMBEDDING_DIM = 128
WORDS = EMBEDDING_DIM // 4
NW = 32
BATCH = 4096
SEQ = 50
B_TOTAL = BATCH * SEQ
BPW = BATCH // NW
PER_W = BPW * SEQ
OUT_TILE = BPW * EMBEDDING_DIM

RP_ROWS = 800
RP_BLOCKS = NUM_EMBEDDINGS // RP_ROWS


def _repack_body(in_ref, out_ref):
    q = pltpu.bitcast(in_ref[...], jnp.int32)
    m = lax.broadcasted_iota(jnp.int32, q.shape, 1) // WORDS
    sh = m * 8
    acc = None
    for k in range(4):
        s_k = jnp.concatenate([q[:, WORDS * k:WORDS * (k + 1)]] * 4, axis=1)
        p_k = lax.shift_right_logical(s_k, sh) & 0xFF
        acc = p_k if k == 0 else acc | (p_k << (8 * k))
    out_ref[...] = acc


_repack = pl.pallas_call(
    _repack_body,
    grid=(RP_BLOCKS,),
    in_specs=[pl.BlockSpec((RP_ROWS, EMBEDDING_DIM), lambda i: (i, 0))],
    out_specs=pl.BlockSpec((RP_ROWS // 4, EMBEDDING_DIM), lambda i: (i, 0)),
    out_shape=jax.ShapeDtypeStruct((NUM_EMBEDDINGS // 4, EMBEDDING_DIM),
                                   jnp.int32),
)


NBUF = 5


def _body(ids_hbm, table_hbm, params_hbm, out_hbm,
          idx_t, rows_v, out_v, lut_v, par_v, *sems):
    c = lax.axis_index("c")
    s_ax = lax.axis_index("s")
    wid = s_ax * 2 + c
    sem_g = sems[:NBUF]
    sem_o = sems[NBUF:]

    pltpu.sync_copy(params_hbm, par_v)
    pltpu.sync_copy(ids_hbm.at[:, pl.ds(pl.multiple_of(wid * BPW, 8), BPW)],
                    idx_t)

    scv = par_v[pl.ds(0, 16)]
    zpv = par_v[pl.ds(16, 16)]
    ii = lax.iota(jnp.int32, 16)

    for k in range(16):
        qv = (ii + 16 * k).astype(jnp.float32)
        lut_v[pl.ds(16 * k, 16)] = (qv - zpv) * scv

    def gather_copy(s, b):
        return pltpu.make_async_copy(
            table_hbm.at[idx_t.at[s]],
            rows_v.at[b], sem_g[b])

    def dequant_tile(b):
        @pl.loop(0, BPW, unroll=4)
        def _(r):
            base = (b * BPW + r) * EMBEDDING_DIM
            for h in range(2):
                w = rows_v[b, r, pl.ds(16 * h, 16)]
                for k in range(4):
                    qb = lax.shift_right_logical(w, k * 8) if k else w
                    if k in (1, 2):
                        val = plsc.load_gather(lut_v, [qb & 0xFF])
                    else:
                        qf = (qb & 0xFF if k == 0 else qb).astype(jnp.float32)
                        val = (qf - zpv) * scv
                    out_v[pl.ds(base + WORDS * k + 16 * h, 16)] = val

    def out_copy(s, b):
        return pltpu.make_async_copy(
            out_v.at[pl.ds(b * OUT_TILE, OUT_TILE)],
            out_hbm.at[pl.ds(
                pl.multiple_of(wid * OUT_TILE, 8) + s * (BATCH * EMBEDDING_DIM),
                OUT_TILE)],
            sem_o[b])

    for p in range(NBUF - 1):
        gather_copy(p, p).start()

    @pl.loop(0, SEQ, step=NBUF)
    def _(s):
        for par in range(NBUF):
            sb = s + par

            @pl.when(sb + NBUF - 1 < SEQ)
            def _():
                gather_copy(sb + NBUF - 1, (par + NBUF - 1) % NBUF).start()

            gather_copy(sb, par).wait()

            @pl.when(sb >= NBUF)
            def _():
                out_copy(sb - NBUF, par).wait()

            dequant_tile(par)
            out_copy(sb, par).start()

    for p in range(NBUF):
        out_copy(SEQ - NBUF + p, p).wait()


_gather_dequant = pl.kernel(
    _body,
    out_type=jax.ShapeDtypeStruct((B_TOTAL * EMBEDDING_DIM,), jnp.float32),
    mesh=plsc.VectorSubcoreMesh(core_axis_name="c", subcore_axis_name="s"),
    compiler_params=pltpu.CompilerParams(needs_layout_passes=False,
                                         use_tc_tiling_on_sc=False),
    scratch_types=[
        pltpu.VMEM((SEQ, BPW), jnp.int32),
        pltpu.VMEM((NBUF, BPW, WORDS), jnp.int32),
        pltpu.VMEM((NBUF * OUT_TILE,), jnp.float32),
        pltpu.VMEM((256,), jnp.float32),
        pltpu.VMEM((32,), jnp.float32),
    ] + [pltpu.SemaphoreType.DMA] * (2 * NBUF),
)


def kernel(input_ids, weight_quantized, scale, zero_point):
    ids = input_ids.T
    table = _repack(weight_quantized).reshape(NUM_EMBEDDINGS, WORDS)
    params = jnp.concatenate([
        jnp.broadcast_to(scale.astype(jnp.float32), (16,)),
        jnp.broadcast_to(zero_point.astype(jnp.float32), (16,)),
    ])
    out = _gather_dequant(ids, table, params)
    return out.reshape(SEQ, BATCH, EMBEDDING_DIM).transpose(1, 0, 2)


# speedup vs baseline: 6.8863x; 2.0955x over previous
import jax
import jax.numpy as jnp
from jax import lax
from jax.experimental import pallas as pl
from jax.experimental.pallas import tpu as pltpu
from jax.experimental.pallas import tpu_sc as plsc

NUM_EMBEDDINGS = 100000
EMBEDDING_DIM = 128
WORDS = EMBEDDING_DIM // 4
NW = 32
BATCH = 4096
SEQ = 50
B_TOTAL = BATCH * SEQ
BPW = BATCH // NW
PER_W = BPW * SEQ
OUT_TILE = BPW * EMBEDDING_DIM

RP_ROWS = 800
RP_BLOCKS = NUM_EMBEDDINGS // RP_ROWS


def _repack_body(in_ref, out_ref):
    q = pltpu.bitcast(in_ref[...], jnp.int32)
    m = lax.broadcasted_iota(jnp.int32, q.shape, 1) // WORDS
    sh = m * 8
    acc = None
    for k in range(4):
        s_k = jnp.concatenate([q[:, WORDS * k:WORDS * (k + 1)]] * 4, axis=1)
        p_k = lax.shift_right_logical(s_k, sh) & 0xFF
        acc = p_k if k == 0 else acc | (p_k << (8 * k))
    out_ref[...] = acc


_repack = pl.pallas_call(
    _repack_body,
    grid=(RP_BLOCKS,),
    in_specs=[pl.BlockSpec((RP_ROWS, EMBEDDING_DIM), lambda i: (i, 0))],
    out_specs=pl.BlockSpec((RP_ROWS // 4, EMBEDDING_DIM), lambda i: (i, 0)),
    out_shape=jax.ShapeDtypeStruct((NUM_EMBEDDINGS // 4, EMBEDDING_DIM),
                                   jnp.int32),
)


NBUF = 5


def _body(ids_hbm, table_hbm, params_hbm, out_hbm,
          idx_t, rows_v, out_v, lut_v, par_v, *sems):
    c = lax.axis_index("c")
    s_ax = lax.axis_index("s")
    wid = s_ax * 2 + c
    sem_g = sems[:NBUF]
    sem_o = sems[NBUF:]

    pltpu.sync_copy(params_hbm, par_v)
    pltpu.sync_copy(ids_hbm.at[:, pl.ds(pl.multiple_of(wid * BPW, 8), BPW)],
                    idx_t)

    scv = par_v[pl.ds(0, 16)]
    zpv = par_v[pl.ds(16, 16)]
    ii = lax.iota(jnp.int32, 16)

    for k in range(16):
        qv = (ii + 16 * k).astype(jnp.float32)
        lut_v[pl.ds(16 * k, 16)] = (qv - zpv) * scv

    def gather_copy(s, b):
        return pltpu.make_async_copy(
            table_hbm.at[idx_t.at[s]],
            rows_v.at[b], sem_g[b])

    def dequant_tile(b):
        @plsc.parallel_loop(0, BPW, unroll=4)
        def _(r):
            base = (b * BPW + r) * EMBEDDING_DIM
            for h in range(2):
                w = rows_v[b, r, pl.ds(16 * h, 16)]
                for k in range(4):
                    qb = lax.shift_right_logical(w, k * 8) if k else w
                    if k in (1, 2):
                        val = plsc.load_gather(lut_v, [qb & 0xFF])
                    else:
                        qf = (qb & 0xFF if k == 0 else qb).astype(jnp.float32)
                        val = (qf - zpv) * scv
                    out_v[pl.ds(base + WORDS * k + 16 * h, 16)] = val

    def out_copy(s, b):
        return pltpu.make_async_copy(
            out_v.at[pl.ds(b * OUT_TILE, OUT_TILE)],
            out_hbm.at[pl.ds(
                pl.multiple_of(wid * OUT_TILE, 8) + s * (BATCH * EMBEDDING_DIM),
                OUT_TILE)],
            sem_o[b])

    for p in range(NBUF - 1):
        gather_copy(p, p).start()

    @pl.loop(0, SEQ, step=NBUF)
    def _(s):
        for par in range(NBUF):
            sb = s + par

            @pl.when(sb + NBUF - 1 < SEQ)
            def _():
                gather_copy(sb + NBUF - 1, (par + NBUF - 1) % NBUF).start()

            gather_copy(sb, par).wait()

            @pl.when(sb >= NBUF)
            def _():
                out_copy(sb - NBUF, par).wait()

            dequant_tile(par)
            out_copy(sb, par).start()

    for p in range(NBUF):
        out_copy(SEQ - NBUF + p, p).wait()


_gather_dequant = pl.kernel(
    _body,
    out_type=jax.ShapeDtypeStruct((B_TOTAL * EMBEDDING_DIM,), jnp.float32),
    mesh=plsc.VectorSubcoreMesh(core_axis_name="c", subcore_axis_name="s"),
    compiler_params=pltpu.CompilerParams(needs_layout_passes=False,
                                         use_tc_tiling_on_sc=False),
    scratch_types=[
        pltpu.VMEM((SEQ, BPW), jnp.int32),
        pltpu.VMEM((NBUF, BPW, WORDS), jnp.int32),
        pltpu.VMEM((NBUF * OUT_TILE,), jnp.float32),
        pltpu.VMEM((256,), jnp.float32),
        pltpu.VMEM((32,), jnp.float32),
    ] + [pltpu.SemaphoreType.DMA] * (2 * NBUF),
)


def kernel(input_ids, weight_quantized, scale, zero_point):
    ids = input_ids.T
    table = _repack(weight_quantized).reshape(NUM_EMBEDDINGS, WORDS)
    params = jnp.concatenate([
        jnp.broadcast_to(scale.astype(jnp.float32), (16,)),
        jnp.broadcast_to(zero_point.astype(jnp.float32), (16,)),
    ])
    out = _gather_dequant(ids, table, params)
    return out.reshape(SEQ, BATCH, EMBEDDING_DIM).transpose(1, 0, 2)
